# Initial kernel scaffold; baseline (speedup 1.0000x reference)
import jax
import jax.numpy as jnp
from jax.experimental import pallas as pl


def kernel(logits_SPT, seq_SP, valid_output_mask_TiTo):
    raise NotImplementedError("write your pallas kernel here")


# speedup vs baseline: 2.8398x; 2.8398x over previous
import functools

import jax
import jax.numpy as jnp
from jax import lax
from jax.experimental import pallas as pl
from jax.experimental.pallas import tpu as pltpu
from jax.experimental.pallas import tpu_sc as plsc

_D = 64
_V = 33
_NC = 2
_NS = 16
_NW = _NC * _NS
_CHUNK = 512


def _make_sc_call(n_rows: int):
    rows_per_w = n_rows // _NW
    n_chunks = rows_per_w // _CHUNK
    mesh = plsc.VectorSubcoreMesh(core_axis_name="c", subcore_axis_name="s")

    @functools.partial(
        pl.kernel,
        out_type=jax.ShapeDtypeStruct((n_rows, _D), jnp.float32),
        mesh=mesh,
        scratch_types=[
            pltpu.VMEM((_V, _D), jnp.float32),
            pltpu.VMEM((_CHUNK,), jnp.int32),
            pltpu.VMEM((_CHUNK, _D), jnp.float32),
        ],
    )
    def sc_kernel(logits_hbm, seq_hbm, mask_hbm, out_hbm, mask_v, idx_v, buf_v):
        wid = lax.axis_index("s") * _NC + lax.axis_index("c")
        base = wid * rows_per_w
        pltpu.sync_copy(mask_hbm, mask_v)

        def chunk_body(ci, carry):
            row0 = base + ci * _CHUNK
            pltpu.sync_copy(logits_hbm.at[pl.ds(row0, _CHUNK)], buf_v)
            pltpu.sync_copy(seq_hbm.at[pl.ds(row0, _CHUNK)], idx_v)

            def row_body(g, c2):
                r0 = g * 16
                t16 = idx_v[pl.ds(r0, 16)]
                for j in range(16):
                    t = t16[j]
                    for q in range(_D // 16):
                        m = mask_v[t, pl.ds(q * 16, 16)]
                        plsc.addupdate(buf_v.at[r0 + j, pl.ds(q * 16, 16)], m)
                return c2

            lax.fori_loop(0, _CHUNK // 16, row_body, 0, unroll=False)
            pltpu.sync_copy(buf_v, out_hbm.at[pl.ds(row0, _CHUNK)])
            return carry

        lax.fori_loop(0, n_chunks, chunk_body, 0, unroll=False)

    return sc_kernel


@jax.jit
def kernel(logits_SPT, seq_SP, valid_output_mask_TiTo):
    S, P, T = logits_SPT.shape
    n_rows = S * P
    logits2d = logits_SPT.reshape(n_rows, T)
    seq1d = seq_SP.reshape(n_rows).astype(jnp.int32)
    mask = valid_output_mask_TiTo.astype(jnp.float32)
    out = _make_sc_call(n_rows)(logits2d, seq1d, mask)
    return out.reshape(S, P, T)
